# baseline (speedup 1.0000x reference)
import functools
import math

import numpy as np

import jax
import jax.numpy as jnp
from jax.experimental import pallas as pl
from jax.experimental.pallas import tpu as pltpu


def _w_interp_matrix_t(in_size, out_size, scale):
    m = np.zeros((out_size, in_size), dtype=np.float32)
    rscale = 1.0 / float(scale)
    for o in range(out_size):
        src = max((o + 0.5) * rscale - 0.5, 0.0)
        i0 = min(int(math.floor(src)), in_size - 1)
        i1 = min(i0 + 1, in_size - 1)
        lam = float(src - i0)
        m[o, i0] += 1.0 - lam
        m[o, i1] += lam
    return m.T


def _h_phases(scale):
    phases = []
    for j in range(scale):
        v = (j + 0.5) / float(scale) - 0.5
        d = int(math.floor(v))
        phases.append((d, float(v - d)))
    return phases


def _shift_rows(a3, d, h):
    if d == 0:
        return a3
    if d < 0:
        pad = [a3[:, :1, :]] * (-d)
        return jnp.concatenate(pad + [a3[:, : h + d, :]], axis=1)
    pad = [a3[:, h - 1 :, :]] * d
    return jnp.concatenate([a3[:, d:, :]] + pad, axis=1)


def _upsample_kernel(ww_ref, x_ref, o_ref, *, tb, h_in, w_out, phases):
    x = x_ref[...]
    a = jnp.dot(x, ww_ref[...], preferred_element_type=jnp.float32)
    a3 = a.reshape(tb, h_in, w_out)
    s = len(phases)
    outs = []
    for d, lam in phases:
        r0 = _shift_rows(a3, d, h_in)
        if lam == 0.0:
            outs.append(r0)
        else:
            r1 = _shift_rows(a3, d + 1, h_in)
            outs.append(r0 * (1.0 - lam) + r1 * lam)
    o_ref[...] = (
        jnp.concatenate(outs, axis=-1)
        .reshape(tb * h_in, s * w_out)
        .astype(o_ref.dtype)
    )


@functools.partial(jax.jit, static_argnames=("scale",))
def _upsample_bilinear(x, scale):
    n, c, h_in, w_in = x.shape
    s = int(scale)
    h_out, w_out = h_in * s, w_in * s
    b = n * c

    ww = jnp.asarray(_w_interp_matrix_t(w_in, w_out, s))
    x2 = x.reshape(b * h_in, w_in)

    tb = 32
    while b % tb:
        tb //= 2
    grid = b // tb

    kern = functools.partial(
        _upsample_kernel, tb=tb, h_in=h_in, w_out=w_out, phases=_h_phases(s)
    )
    out2 = pl.pallas_call(
        kern,
        out_shape=jax.ShapeDtypeStruct((b * h_in, s * w_out), x.dtype),
        grid=(grid,),
        in_specs=[
            pl.BlockSpec((w_in, w_out), lambda i: (0, 0)),
            pl.BlockSpec((tb * h_in, w_in), lambda i: (i, 0)),
        ],
        out_specs=pl.BlockSpec((tb * h_in, s * w_out), lambda i: (i, 0)),
        compiler_params=pltpu.CompilerParams(
            dimension_semantics=("parallel",),
        ),
    )(ww, x2)

    return out2.reshape(n, c, h_out, w_out)


def kernel(x):
    return _upsample_bilinear(x, 2)


# speedup vs baseline: 1.1227x; 1.1227x over previous
import functools
import math

import numpy as np

import jax
import jax.numpy as jnp
from jax.experimental import pallas as pl
from jax.experimental.pallas import tpu as pltpu


def _w_interp_matrix_t(in_size, out_size, scale):
    m = np.zeros((out_size, in_size), dtype=np.float32)
    rscale = 1.0 / float(scale)
    for o in range(out_size):
        src = max((o + 0.5) * rscale - 0.5, 0.0)
        i0 = min(int(math.floor(src)), in_size - 1)
        i1 = min(i0 + 1, in_size - 1)
        lam = float(src - i0)
        m[o, i0] += 1.0 - lam
        m[o, i1] += lam
    return m.T


def _h_phases(scale):
    phases = []
    for j in range(scale):
        v = (j + 0.5) / float(scale) - 0.5
        d = int(math.floor(v))
        phases.append((d, float(v - d)))
    return phases


def _shift_rows(a3, d, h):
    if d == 0:
        return a3
    if d < 0:
        pad = [a3[:, :1, :]] * (-d)
        return jnp.concatenate(pad + [a3[:, : h + d, :]], axis=1)
    pad = [a3[:, h - 1 :, :]] * d
    return jnp.concatenate([a3[:, d:, :]] + pad, axis=1)


def _upsample_kernel(ww_ref, x_ref, o_ref, *, tb, h_in, w_out, phases):
    x = x_ref[...]
    a = jnp.dot(x, ww_ref[...], preferred_element_type=jnp.float32)
    a3 = a.reshape(tb, h_in, w_out)
    s = len(phases)
    outs = []
    for d, lam in phases:
        r0 = _shift_rows(a3, d, h_in)
        if lam == 0.0:
            outs.append(r0)
        else:
            r1 = _shift_rows(a3, d + 1, h_in)
            outs.append(r0 * (1.0 - lam) + r1 * lam)
    o_ref[...] = (
        jnp.concatenate(outs, axis=-1)
        .reshape(tb * h_in, s * w_out)
        .astype(o_ref.dtype)
    )


@functools.partial(jax.jit, static_argnames=("scale",))
def _upsample_bilinear(x, scale):
    n, c, h_in, w_in = x.shape
    s = int(scale)
    h_out, w_out = h_in * s, w_in * s
    b = n * c

    ww = jnp.asarray(_w_interp_matrix_t(w_in, w_out, s))
    x2 = x.reshape(b * h_in, w_in)

    tb = 256
    while b % tb:
        tb //= 2
    grid = b // tb

    kern = functools.partial(
        _upsample_kernel, tb=tb, h_in=h_in, w_out=w_out, phases=_h_phases(s)
    )
    out2 = pl.pallas_call(
        kern,
        out_shape=jax.ShapeDtypeStruct((b * h_in, s * w_out), x.dtype),
        grid=(grid,),
        in_specs=[
            pl.BlockSpec((w_in, w_out), lambda i: (0, 0)),
            pl.BlockSpec((tb * h_in, w_in), lambda i: (i, 0)),
        ],
        out_specs=pl.BlockSpec((tb * h_in, s * w_out), lambda i: (i, 0)),
        compiler_params=pltpu.CompilerParams(
            dimension_semantics=("parallel",),
        ),
    )(ww, x2)

    return out2.reshape(n, c, h_out, w_out)


def kernel(x):
    return _upsample_bilinear(x, 2)


# speedup vs baseline: 1.1260x; 1.0030x over previous
import functools
import math

import numpy as np

import jax
import jax.numpy as jnp
from jax.experimental import pallas as pl
from jax.experimental.pallas import tpu as pltpu


def _w_interp_matrix_t(in_size, out_size, scale):
    m = np.zeros((out_size, in_size), dtype=np.float32)
    rscale = 1.0 / float(scale)
    for o in range(out_size):
        src = max((o + 0.5) * rscale - 0.5, 0.0)
        i0 = min(int(math.floor(src)), in_size - 1)
        i1 = min(i0 + 1, in_size - 1)
        lam = float(src - i0)
        m[o, i0] += 1.0 - lam
        m[o, i1] += lam
    return m.T


def _h_phases(scale):
    phases = []
    for j in range(scale):
        v = (j + 0.5) / float(scale) - 0.5
        d = int(math.floor(v))
        phases.append((d, float(v - d)))
    return phases


def _shift_rows(a3, d, h):
    if d == 0:
        return a3
    if d < 0:
        pad = [a3[:, :1, :]] * (-d)
        return jnp.concatenate(pad + [a3[:, : h + d, :]], axis=1)
    pad = [a3[:, h - 1 :, :]] * d
    return jnp.concatenate([a3[:, d:, :]] + pad, axis=1)


def _upsample_kernel(ww_ref, x_ref, o_ref, *, tb, h_in, w_out, phases):
    x = x_ref[...]
    a = jnp.dot(x, ww_ref[...], preferred_element_type=jnp.float32)
    a3 = a.reshape(tb, h_in, w_out)
    s = len(phases)
    outs = []
    for d, lam in phases:
        r0 = _shift_rows(a3, d, h_in)
        if lam == 0.0:
            outs.append(r0)
        else:
            r1 = _shift_rows(a3, d + 1, h_in)
            outs.append(r0 * (1.0 - lam) + r1 * lam)
    o_ref[...] = (
        jnp.concatenate(outs, axis=-1)
        .reshape(tb * h_in, s * w_out)
        .astype(o_ref.dtype)
    )


@functools.partial(jax.jit, static_argnames=("scale",))
def _upsample_bilinear(x, scale):
    n, c, h_in, w_in = x.shape
    s = int(scale)
    h_out, w_out = h_in * s, w_in * s
    b = n * c

    ww = jnp.asarray(_w_interp_matrix_t(w_in, w_out, s))
    x2 = x.reshape(b * h_in, w_in)

    tb = 256
    while b % tb:
        tb //= 2
    grid = b // tb

    kern = functools.partial(
        _upsample_kernel, tb=tb, h_in=h_in, w_out=w_out, phases=_h_phases(s)
    )
    out2 = pl.pallas_call(
        kern,
        out_shape=jax.ShapeDtypeStruct((b * h_in, s * w_out), x.dtype),
        grid=(grid,),
        in_specs=[
            pl.BlockSpec((w_in, w_out), lambda i: (0, 0)),
            pl.BlockSpec((tb * h_in, w_in), lambda i: (i, 0)),
        ],
        out_specs=pl.BlockSpec((tb * h_in, s * w_out), lambda i: (i, 0)),
        compiler_params=pltpu.CompilerParams(
            dimension_semantics=("parallel",),
            vmem_limit_bytes=60 * 1024 * 1024,
        ),
    )(ww, x2)

    return out2.reshape(n, c, h_out, w_out)


def kernel(x):
    return _upsample_bilinear(x, 2)
